# baseline (speedup 1.0000x reference)
import functools
import jax
import jax.numpy as jnp
from jax import lax
from jax.experimental import pallas as pl
from jax.experimental.pallas import tpu as pltpu
from jax.experimental.pallas import tpu_sc as plsc

F32 = jnp.float32
D = 128
ADP = 128
CATW = D + ADP
NW = 32
CH = 80


def _pad_rows(n):
    q = NW * CH
    return ((n + q - 1) // q) * q


def _t1_body(wdx_ref, wrx_ref, dxa1_ref, dxa2_ref, dxb_ref,
             rxa1_ref, rxa2_ref, rxb_ref,
             p1dx_ref, catdx_ref, p1rx_ref, catrx_ref):
    wdx = wdx_ref[...]
    p1dx_ref[...] = jnp.dot(wdx, dxa1_ref[...], preferred_element_type=F32)
    catdx_ref[:, :D] = wdx
    catdx_ref[:, D:] = jnp.dot(wdx, dxa2_ref[...], preferred_element_type=F32) + dxb_ref[...]
    wrx = wrx_ref[...]
    p1rx_ref[...] = jnp.dot(wrx, rxa1_ref[...], preferred_element_type=F32)
    catrx_ref[:, :D] = wrx
    catrx_ref[:, D:] = jnp.dot(wrx, rxa2_ref[...], preferred_element_type=F32) + rxb_ref[...]


def _t1(wdx, wrx, dxa1, dxa2, dxb, rxa1, rxa2, rxb):
    vdx, vrx = wdx.shape[0], wrx.shape[0]
    return pl.pallas_call(
        _t1_body,
        out_shape=[
            jax.ShapeDtypeStruct((vdx, ADP), F32),
            jax.ShapeDtypeStruct((vdx, CATW), F32),
            jax.ShapeDtypeStruct((vrx, ADP), F32),
            jax.ShapeDtypeStruct((vrx, CATW), F32),
        ],
    )(wdx, wrx, dxa1, dxa2, dxb, rxa1, rxa2, rxb)


def _sc_gather_body(widths, chunks, *refs):
    n = len(widths)
    tabs = refs[:n]
    idxs = refs[n:2 * n]
    outs = refs[2 * n:3 * n]
    scratch = refs[3 * n:]
    sem = scratch[-1]
    wid = lax.axis_index("s") * 2 + lax.axis_index("c")
    for i in range(n):
        idx_v = scratch[2 * i]
        rows_v = scratch[2 * i + 1]
        base = wid * (chunks[i] * CH)

        def body(c, _, idx_hbm=idxs[i], tab_hbm=tabs[i], out_hbm=outs[i],
                 idx_v=idx_v, rows_v=rows_v, base=base):
            off = base + c * CH
            pltpu.sync_copy(idx_hbm.at[pl.ds(off, CH)], idx_v)
            pltpu.async_copy(tab_hbm.at[idx_v], rows_v, sem).wait()
            pltpu.sync_copy(rows_v, out_hbm.at[pl.ds(off, CH)])
            return _

        lax.fori_loop(0, chunks[i], body, 0)


def _sc_gathers(tables, idx_lists):
    widths = tuple(int(t.shape[1]) for t in tables)
    chunks = tuple(int(ix.shape[0]) // (NW * CH) for ix in idx_lists)
    scratch = []
    for w in widths:
        scratch.append(pltpu.VMEM((CH,), jnp.int32))
        scratch.append(pltpu.VMEM((CH, w), F32))
    scratch.append(pltpu.SemaphoreType.DMA)
    out_type = [jax.ShapeDtypeStruct((int(ix.shape[0]), w), F32)
                for ix, w in zip(idx_lists, widths)]
    mesh = plsc.VectorSubcoreMesh(core_axis_name="c", subcore_axis_name="s")
    k = pl.kernel(
        functools.partial(_sc_gather_body, widths, chunks),
        out_type=out_type,
        mesh=mesh,
        scratch_types=scratch,
    )
    return k(*tables, *idx_lists)


def _t2_body(g1_ref, cat_ref, u_ref, out_ref):
    g1 = g1_ref[...]
    cat = cat_ref[...]
    ea = cat[:, :, :D]
    p2 = cat[:, :, D:]
    t = jnp.tanh(g1 + p2)
    pre = jnp.sum(t * u_ref[...], axis=2)
    m = jnp.max(pre, axis=1, keepdims=True)
    e = jnp.exp(pre - m)
    attn = e / jnp.sum(e, axis=1, keepdims=True)
    out_ref[...] = jnp.sum(attn[:, :, None] * ea, axis=1)


def _t2(g1, cat, u, maxa, lb=256):
    nl = g1.shape[0] // maxa
    g1 = g1.reshape(nl, maxa, ADP)
    cat = cat.reshape(nl, maxa, CATW)
    grid = nl // lb
    return pl.pallas_call(
        _t2_body,
        grid=(grid,),
        in_specs=[
            pl.BlockSpec((lb, maxa, ADP), lambda i: (i, 0, 0)),
            pl.BlockSpec((lb, maxa, CATW), lambda i: (i, 0, 0)),
            pl.BlockSpec((1, 1, ADP), lambda i: (0, 0, 0)),
        ],
        out_specs=pl.BlockSpec((lb, D), lambda i: (i, 0)),
        out_shape=jax.ShapeDtypeStruct((nl, D), F32),
    )(g1, cat, u.reshape(1, 1, ADP))


def _t3_body(dx_ref, rx_ref, out_ref):
    sdx = jnp.sum(dx_ref[...][:, :, :D], axis=1)
    srx = jnp.sum(rx_ref[...][:, :, :D], axis=1)
    ndx = sdx * lax.rsqrt(jnp.maximum(jnp.sum(sdx * sdx, axis=1, keepdims=True), 1e-24))
    nrx = srx * lax.rsqrt(jnp.maximum(jnp.sum(srx * srx, axis=1, keepdims=True), 1e-24))
    out_ref[...] = ndx + nrx


def _t3(gdx, grx, ndx, nrx, rows, rb=64):
    gdx = gdx.reshape(rows, ndx, 2 * D)
    grx = grx.reshape(rows, nrx, 2 * D)
    return pl.pallas_call(
        _t3_body,
        grid=(rows // rb,),
        in_specs=[
            pl.BlockSpec((rb, ndx, 2 * D), lambda i: (i, 0, 0)),
            pl.BlockSpec((rb, nrx, 2 * D), lambda i: (i, 0, 0)),
        ],
        out_specs=pl.BlockSpec((rb, D), lambda i: (i, 0)),
        out_shape=jax.ShapeDtypeStruct((rows, D), F32),
    )(gdx, grx)


def _t4_body(x_ref, w_ref, b_ref, out_ref):
    y = jnp.dot(x_ref[...], w_ref[...], preferred_element_type=F32) + b_ref[...]
    m = jnp.max(y, axis=1, keepdims=True)
    e = jnp.exp(y - m)
    out_ref[...] = e / jnp.sum(e, axis=1, keepdims=True)


def _t4(x, w, b, rb=64):
    rows, nv = x.shape[0], w.shape[1]
    return pl.pallas_call(
        _t4_body,
        grid=(rows // rb,),
        in_specs=[
            pl.BlockSpec((rb, D), lambda i: (i, 0)),
            pl.BlockSpec((D, nv), lambda i: (0, 0)),
            pl.BlockSpec((1, nv), lambda i: (0, 0)),
        ],
        out_specs=pl.BlockSpec((rb, nv), lambda i: (i, 0)),
        out_shape=jax.ShapeDtypeStruct((rows, nv), F32),
    )(x, w, b.reshape(1, nv))


def _t5_body(oh_ref, tab_ref, out_ref):
    out_ref[0] = jnp.dot(oh_ref[0], tab_ref[...], preferred_element_type=F32)


def _t5(onehot, table):
    v, b, nv = onehot.shape
    return pl.pallas_call(
        _t5_body,
        grid=(v,),
        in_specs=[
            pl.BlockSpec((1, b, nv), lambda i: (i, 0, 0)),
            pl.BlockSpec((nv, D), lambda i: (0, 0)),
        ],
        out_specs=pl.BlockSpec((1, b, D), lambda i: (i, 0, 0)),
        out_shape=jax.ShapeDtypeStruct((v, b, D), F32),
    )(onehot, table)


def kernel(dxseqs, drugseqs, dx_onehot, drug_onehot, dxLeavesList,
           dxAncestorsList, drugLeavesList, drugAncestorsList,
           ctd_dx_leaves_list, ctd_dx_ancesster_list, ctd_dx_rel_list,
           ctd_dx_permute_list, ctd_rx_leaves_list, ctd_rx_ancesster_list,
           ctd_rx_rel_list, ctd_rx_permute_list, EHRdxEmb_W, EHRdrugEmb_W,
           dxOntoW, drugOntoW, dxAttnW, dxAttnb, dxAttnU, drugAttnW,
           drugAttnb, drugAttnU, cooccurW, cooccurB):
    B, V, NDX = dxseqs.shape
    NRX = drugseqs.shape[2]
    DXV, MAXA = dxLeavesList.shape
    RXV = drugLeavesList.shape[0]
    AD = dxAttnW.shape[1]

    def padw(m):
        return jnp.pad(m, ((0, 0), (0, ADP - AD)))

    dxa1, dxa2 = padw(dxAttnW[:D]), padw(dxAttnW[D:])
    rxa1, rxa2 = padw(drugAttnW[:D]), padw(drugAttnW[D:])
    dxb = jnp.pad(dxAttnb, (0, ADP - AD)).reshape(1, ADP)
    rxb = jnp.pad(drugAttnb, (0, ADP - AD)).reshape(1, ADP)
    dxu = jnp.pad(dxAttnU[:, 0], (0, ADP - AD))
    rxu = jnp.pad(drugAttnU[:, 0], (0, ADP - AD))

    p1dx, catdx, p1rx, catrx = _t1(dxOntoW, drugOntoW, dxa1, dxa2, dxb,
                                   rxa1, rxa2, rxb)

    def flatpad(ix, n):
        f = ix.reshape(-1).astype(jnp.int32)
        return jnp.pad(f, (0, n - f.shape[0]))

    npair_dx = _pad_rows(DXV * MAXA)
    npair_rx = _pad_rows(RXV * MAXA)
    g1dx, gcatdx, g1rx, gcatrx = _sc_gathers(
        [p1dx, catdx, p1rx, catrx],
        [flatpad(dxLeavesList, npair_dx), flatpad(dxAncestorsList, npair_dx),
         flatpad(drugLeavesList, npair_rx), flatpad(drugAncestorsList, npair_rx)],
    )

    dxall_core = _t2(g1dx, gcatdx, dxu, MAXA)[:DXV]
    rxall_core = _t2(g1rx, gcatrx, rxu, MAXA)[:RXV]

    zrow = jnp.zeros((1, D), F32)
    dx_cat_tab = jnp.concatenate(
        [EHRdxEmb_W, jnp.concatenate([dxall_core, zrow], axis=0)], axis=1)
    rx_cat_tab = jnp.concatenate(
        [EHRdrugEmb_W, jnp.concatenate([rxall_core, zrow], axis=0)], axis=1)

    nseq_dx = _pad_rows(B * V * NDX)
    nseq_rx = _pad_rows(B * V * NRX)
    gseq_dx, gseq_rx = _sc_gathers(
        [dx_cat_tab, rx_cat_tab],
        [flatpad(dxseqs, nseq_dx), flatpad(drugseqs, nseq_rx)],
    )
    gseq_dx = gseq_dx[:B * V * NDX]
    gseq_rx = gseq_rx[:B * V * NRX]

    EHRVEmb = _t3(gseq_dx, gseq_rx, NDX, NRX, B * V)

    cooccurU = _t4(EHRVEmb, cooccurW, cooccurB).reshape(B, V, -1)

    dxontoV = _t5(dx_onehot, dxall_core)
    rxontoV = _t5(drug_onehot, rxall_core)

    ontoEmb = jnp.concatenate(
        [gseq_dx[:, D:].reshape(B, V, NDX, D),
         gseq_rx[:, D:].reshape(B, V, NRX, D)], axis=2)

    return (cooccurU,
            EHRVEmb.reshape(B, V, D),
            ontoEmb,
            jnp.transpose(dxontoV, (1, 0, 2)),
            jnp.transpose(rxontoV, (1, 0, 2)))
